# baseline (speedup 1.0000x reference)
import functools

import jax
import jax.numpy as jnp
from jax import lax
from jax.experimental import pallas as pl
from jax.experimental.pallas import tpu as pltpu
from jax.experimental.pallas import tpu_sc as plsc

_B = 4
_S = 4096
_D = 4096
_NW = 32
_S_PER_W = _S // _NW
_CHUNK = 8
_N_CHUNKS = _S_PER_W // _CHUNK
_SCALE = 64.0
_LANES = 16


def _body(ids_hbm, table_hbm, pos_hbm, out_hbm, idx_v, pos_v, rows_v, sem):
    wid = lax.axis_index("s") * 2 + lax.axis_index("c")
    s0 = wid * _S_PER_W

    for b in range(_B):
        pltpu.sync_copy(ids_hbm.at[b, pl.ds(s0, _S_PER_W)], idx_v.at[b])

    def chunk_body(c, carry):
        s_chunk = s0 + c * _CHUNK
        pltpu.sync_copy(pos_hbm.at[pl.ds(s_chunk, _CHUNK), :], pos_v)
        for b in range(_B):
            pltpu.async_copy(
                table_hbm.at[idx_v.at[b, pl.ds(c * _CHUNK, _CHUNK)]],
                rows_v,
                sem,
            ).wait()

            def fma(j, acc):
                off = j * _LANES
                for r in range(_CHUNK):
                    rows_v[r, pl.ds(off, _LANES)] = (
                        rows_v[r, pl.ds(off, _LANES)] * _SCALE
                        + pos_v[r, pl.ds(off, _LANES)]
                    )
                return acc

            lax.fori_loop(0, _D // _LANES, fma, 0)

            pltpu.sync_copy(rows_v, out_hbm.at[b, pl.ds(s_chunk, _CHUNK), :])
        return carry

    lax.fori_loop(0, _N_CHUNKS, chunk_body, 0)


@jax.jit
def _embed(token_ids, token_table, pos_table):
    mesh = plsc.VectorSubcoreMesh(core_axis_name="c", subcore_axis_name="s")
    return pl.kernel(
        _body,
        out_type=jax.ShapeDtypeStruct((_B, _S, _D), jnp.float32),
        mesh=mesh,
        scratch_types=[
            pltpu.VMEM((_B, _S_PER_W), jnp.int32),
            pltpu.VMEM((_CHUNK, _D), jnp.float32),
            pltpu.VMEM((_CHUNK, _D), jnp.float32),
            pltpu.SemaphoreType.DMA,
        ],
    )(token_ids, token_table, pos_table)


def kernel(token_ids, token_table, pos_table):
    return _embed(token_ids.astype(jnp.int32), token_table, pos_table)
